# baseline (speedup 1.0000x reference)
import functools

import jax
import jax.numpy as jnp
from jax.experimental import pallas as pl
from jax.experimental.pallas import tpu as pltpu

_T, _D, _F, _E, _K = 2048, 1024, 2048, 8, 2
_B = 512
_A = _T * _K
_NB = _A // _B + _E - 1
_NPAD = _NB * _B
_NCH = _T // _B


def _router_body(x_ref, wr_ref, pos_ref, gates_ref, toks_ref, be_ref, valid_ref,
                 oh_ref):
    x = x_ref[...]
    logits = jnp.dot(x, wr_ref[...], preferred_element_type=jnp.float32)
    iota = jax.lax.broadcasted_iota(jnp.int32, logits.shape, 1)
    m1 = jnp.max(logits, axis=-1, keepdims=True)
    i1 = jnp.argmax(logits, axis=-1)[:, None]
    masked = jnp.where(iota == i1, -jnp.inf, logits)
    m2 = jnp.max(masked, axis=-1, keepdims=True)
    i2 = jnp.argmax(masked, axis=-1)[:, None]
    z = jnp.exp(m2 - m1)
    g1 = 1.0 / (1.0 + z)
    g2 = z / (1.0 + z)
    oh1 = (iota == i1).astype(jnp.float32)
    oh2 = (iota == i2).astype(jnp.float32)
    oh_ref[0:_T, :] = oh1
    oh_ref[_T:_A, :] = oh2
    gates_ref[0:_T, :] = g1
    gates_ref[_T:_A, :] = g2
    toks_ref[...] = jax.lax.broadcasted_iota(jnp.int32, (_A, 1), 0) % _T

    n_e = jnp.sum(oh1, axis=0, keepdims=True) + jnp.sum(oh2, axis=0, keepdims=True)
    pc = jnp.ceil(n_e * (1.0 / _B)) * float(_B)
    eiota = jax.lax.broadcasted_iota(jnp.int32, (_E, _E), 0)
    ejota = jax.lax.broadcasted_iota(jnp.int32, (_E, _E), 1)
    strict_upper = (eiota < ejota).astype(jnp.float32)
    ps = jnp.dot(pc, strict_upper, preferred_element_type=jnp.float32)

    ri = jax.lax.broadcasted_iota(jnp.int32, (_B, _B), 0)
    ci = jax.lax.broadcasted_iota(jnp.int32, (_B, _B), 1)
    tri_s = (ci < ri).astype(jnp.float32)

    def body(c, run):
        ohc = oh_ref[pl.ds(c * _B, _B), :]
        local = jnp.dot(tri_s, ohc, preferred_element_type=jnp.float32)
        slot = jnp.sum((ps + run + local) * ohc, axis=1, keepdims=True)
        pos_ref[pl.ds(c * _B, _B), :] = slot.astype(jnp.int32)
        return run + jnp.sum(ohc, axis=0, keepdims=True)

    jax.lax.fori_loop(0, _A // _B, body, jnp.zeros((1, _E), jnp.float32))

    bs = jax.lax.broadcasted_iota(jnp.int32, (128, 1), 0).astype(jnp.float32) * float(_B)
    cnt = jnp.dot((bs >= ps).astype(jnp.float32), jnp.ones((_E, 1), jnp.float32),
                  preferred_element_type=jnp.float32)
    be_ref[...] = (cnt - 1.0).astype(jnp.int32)
    total_pad = jnp.sum(pc)
    valid_ref[...] = (bs < total_pad).astype(jnp.int32)


def _expert_body(sp_ref, xs_ref, gp_ref, wg_ref, wu_ref, wd_ref, ys_ref):
    b = pl.program_id(0)

    @pl.when(sp_ref[1, b] == 1)
    def _compute():
        x = xs_ref[...].astype(jnp.bfloat16)
        g = jnp.dot(x, wg_ref[0], preferred_element_type=jnp.float32)
        u = jnp.dot(x, wu_ref[0], preferred_element_type=jnp.float32)
        h = (g * jax.lax.logistic(g)) * u * gp_ref[...]
        ys_ref[...] = jnp.dot(h.astype(jnp.bfloat16), wd_ref[0],
                              preferred_element_type=jnp.float32)


def kernel(x_TD, w_router_DE, w_gate_EDF, w_up_EDF, w_down_EFD):
    pos, gates, toks, be128, valid128 = pl.pallas_call(
        _router_body,
        out_shape=(
            jax.ShapeDtypeStruct((_A, 1), jnp.int32),
            jax.ShapeDtypeStruct((_A, 1), jnp.float32),
            jax.ShapeDtypeStruct((_A, 1), jnp.int32),
            jax.ShapeDtypeStruct((128, 1), jnp.int32),
            jax.ShapeDtypeStruct((128, 1), jnp.int32),
        ),
        scratch_shapes=[pltpu.VMEM((_A, _E), jnp.float32)],
    )(x_TD, w_router_DE)

    sp = jnp.concatenate([be128[:_NB, 0][None, :], valid128[:_NB, 0][None, :]],
                         axis=0)

    posf = pos[:, 0]
    xs = jnp.zeros((_NPAD, _D), x_TD.dtype).at[posf].set(x_TD[toks[:, 0]])
    gate_pad = jnp.zeros((_NPAD, 1), jnp.float32).at[posf, 0].set(gates[:, 0])

    wg_bf = w_gate_EDF.astype(jnp.bfloat16)
    wu_bf = w_up_EDF.astype(jnp.bfloat16)
    wd_bf = w_down_EFD.astype(jnp.bfloat16)

    ys = pl.pallas_call(
        _expert_body,
        grid_spec=pltpu.PrefetchScalarGridSpec(
            num_scalar_prefetch=1,
            grid=(_NB,),
            in_specs=[
                pl.BlockSpec((_B, _D), lambda b, sp: (b, 0)),
                pl.BlockSpec((_B, 1), lambda b, sp: (b, 0)),
                pl.BlockSpec((1, _D, _F), lambda b, sp: (sp[0, b], 0, 0)),
                pl.BlockSpec((1, _D, _F), lambda b, sp: (sp[0, b], 0, 0)),
                pl.BlockSpec((1, _F, _D), lambda b, sp: (sp[0, b], 0, 0)),
            ],
            out_specs=pl.BlockSpec((_B, _D), lambda b, sp: (b, 0)),
        ),
        out_shape=jax.ShapeDtypeStruct((_NPAD, _D), jnp.float32),
    )(sp, xs, gate_pad, wg_bf, wu_bf, wd_bf)

    out = ys[posf[:_T]] + ys[posf[_T:]]
    return out


# speedup vs baseline: 1.2351x; 1.2351x over previous
import functools

import jax
import jax.numpy as jnp
from jax import lax
from jax.experimental import pallas as pl
from jax.experimental.pallas import tpu as pltpu
from jax.experimental.pallas import tpu_sc as plsc

_T, _D, _F, _E, _K = 2048, 1024, 2048, 8, 2
_B = 512
_A = _T * _K
_NB = _A // _B + _E - 1
_NPAD = _NB * _B

_NC = 2
_NS = 16
_NW = _NC * _NS
_APW = _A // _NW
_GCH = 64
_TPW = _T // _NW
_CCH = 32
_L = 16

_sc_mesh = plsc.VectorSubcoreMesh(core_axis_name="c", subcore_axis_name="s")


def _router_body(x_ref, wr_ref, pos_ref, gates_ref, toks_ref, be_ref, valid_ref,
                 oh_ref):
    x = x_ref[...]
    logits = jnp.dot(x, wr_ref[...], preferred_element_type=jnp.float32)
    iota = jax.lax.broadcasted_iota(jnp.int32, logits.shape, 1)
    m1 = jnp.max(logits, axis=-1, keepdims=True)
    i1 = jnp.argmax(logits, axis=-1)[:, None]
    masked = jnp.where(iota == i1, -jnp.inf, logits)
    m2 = jnp.max(masked, axis=-1, keepdims=True)
    i2 = jnp.argmax(masked, axis=-1)[:, None]
    z = jnp.exp(m2 - m1)
    g1 = 1.0 / (1.0 + z)
    g2 = z / (1.0 + z)
    oh1 = (iota == i1).astype(jnp.float32)
    oh2 = (iota == i2).astype(jnp.float32)
    oh_ref[0:_T, :] = oh1
    oh_ref[_T:_A, :] = oh2
    gates_ref[0:_T, :] = g1
    gates_ref[_T:_A, :] = g2
    toks_ref[...] = jax.lax.broadcasted_iota(jnp.int32, (_A, 1), 0) % _T

    n_e = jnp.sum(oh1, axis=0, keepdims=True) + jnp.sum(oh2, axis=0, keepdims=True)
    pc = jnp.ceil(n_e * (1.0 / _B)) * float(_B)
    eiota = jax.lax.broadcasted_iota(jnp.int32, (_E, _E), 0)
    ejota = jax.lax.broadcasted_iota(jnp.int32, (_E, _E), 1)
    strict_upper = (eiota < ejota).astype(jnp.float32)
    ps = jnp.dot(pc, strict_upper, preferred_element_type=jnp.float32)

    ri = jax.lax.broadcasted_iota(jnp.int32, (_B, _B), 0)
    ci = jax.lax.broadcasted_iota(jnp.int32, (_B, _B), 1)
    tri_s = (ci < ri).astype(jnp.float32)

    def body(c, run):
        ohc = oh_ref[pl.ds(c * _B, _B), :]
        local = jnp.dot(tri_s, ohc, preferred_element_type=jnp.float32)
        slot = jnp.sum((ps + run + local) * ohc, axis=1, keepdims=True)
        pos_ref[pl.ds(c * _B, _B), :] = slot.astype(jnp.int32)
        return run + jnp.sum(ohc, axis=0, keepdims=True)

    jax.lax.fori_loop(0, _A // _B, body, jnp.zeros((1, _E), jnp.float32))

    bs = jax.lax.broadcasted_iota(jnp.int32, (128, 1), 0).astype(jnp.float32) * float(_B)
    cnt = jnp.dot((bs >= ps).astype(jnp.float32), jnp.ones((_E, 1), jnp.float32),
                  preferred_element_type=jnp.float32)
    be_ref[...] = (cnt - 1.0).astype(jnp.int32)
    total_pad = jnp.sum(pc)
    valid_ref[...] = (bs < total_pad).astype(jnp.int32)


@functools.partial(
    pl.kernel,
    mesh=_sc_mesh,
    out_type=jax.ShapeDtypeStruct((_NPAD, _D), jnp.float32),
    scratch_types=[
        pltpu.VMEM((_GCH,), jnp.int32),
        pltpu.VMEM((_GCH,), jnp.int32),
        pltpu.VMEM((_GCH, _D), jnp.float32),
        pltpu.SemaphoreType.DMA,
    ],
)
def _sc_dispatch(x_hbm, toks_hbm, pos_hbm, xs_hbm, idx_v, dst_v, rows_v, sem):
    wid = lax.axis_index("s") * _NC + lax.axis_index("c")
    for c in range(_APW // _GCH):
        base = wid * _APW + c * _GCH
        pltpu.sync_copy(toks_hbm.at[pl.ds(base, _GCH)], idx_v)
        pltpu.sync_copy(pos_hbm.at[pl.ds(base, _GCH)], dst_v)
        pltpu.async_copy(x_hbm.at[idx_v], rows_v, sem).wait()
        pltpu.async_copy(rows_v, xs_hbm.at[dst_v], sem).wait()


def _expert_body(sp_ref, xs_ref, wg_ref, wu_ref, wd_ref, ys_ref):
    b = pl.program_id(0)

    @pl.when(sp_ref[1, b] == 1)
    def _compute():
        x = xs_ref[...].astype(jnp.bfloat16)
        g = jnp.dot(x, wg_ref[0], preferred_element_type=jnp.float32)
        u = jnp.dot(x, wu_ref[0], preferred_element_type=jnp.float32)
        h = (g * jax.lax.logistic(g)) * u
        ys_ref[...] = jnp.dot(h.astype(jnp.bfloat16), wd_ref[0],
                              preferred_element_type=jnp.float32)


@functools.partial(
    pl.kernel,
    mesh=_sc_mesh,
    out_type=jax.ShapeDtypeStruct((_T, _D), jnp.float32),
    scratch_types=[
        pltpu.VMEM((_CCH,), jnp.int32),
        pltpu.VMEM((_CCH,), jnp.int32),
        pltpu.VMEM((_CCH + _L,), jnp.float32),
        pltpu.VMEM((_CCH + _L,), jnp.float32),
        pltpu.VMEM((_CCH, _D), jnp.float32),
        pltpu.VMEM((_CCH, _D), jnp.float32),
        pltpu.SemaphoreType.DMA,
    ],
)
def _sc_combine(ys_hbm, pos_hbm, gates_hbm, out_hbm,
                p0_v, p1_v, g0_v, g1_v, buf0, buf1, sem):
    wid = lax.axis_index("s") * _NC + lax.axis_index("c")
    for c in range(_TPW // _CCH):
        tbase = wid * _TPW + c * _CCH
        pltpu.sync_copy(pos_hbm.at[pl.ds(tbase, _CCH)], p0_v)
        pltpu.sync_copy(pos_hbm.at[pl.ds(_T + tbase, _CCH)], p1_v)
        pltpu.sync_copy(gates_hbm.at[pl.ds(tbase, _CCH)], g0_v.at[pl.ds(0, _CCH)])
        pltpu.sync_copy(gates_hbm.at[pl.ds(_T + tbase, _CCH)], g1_v.at[pl.ds(0, _CCH)])
        pltpu.async_copy(ys_hbm.at[p0_v], buf0, sem).wait()
        pltpu.async_copy(ys_hbm.at[p1_v], buf1, sem).wait()

        def row_body(r, carry):
            ga = g0_v[pl.ds(r, _L)][0]
            gb = g1_v[pl.ds(r, _L)][0]

            def col_body(j, carry2):
                a = buf0[r, pl.ds(j * _L, _L)]
                b = buf1[r, pl.ds(j * _L, _L)]
                buf0[r, pl.ds(j * _L, _L)] = ga * a + gb * b
                return carry2

            return lax.fori_loop(0, _D // _L, col_body, carry)

        lax.fori_loop(0, _CCH, row_body, 0)
        pltpu.sync_copy(buf0, out_hbm.at[pl.ds(tbase, _CCH)])


def kernel(x_TD, w_router_DE, w_gate_EDF, w_up_EDF, w_down_EFD):
    pos, gates, toks, be128, valid128 = pl.pallas_call(
        _router_body,
        out_shape=(
            jax.ShapeDtypeStruct((_A, 1), jnp.int32),
            jax.ShapeDtypeStruct((_A, 1), jnp.float32),
            jax.ShapeDtypeStruct((_A, 1), jnp.int32),
            jax.ShapeDtypeStruct((128, 1), jnp.int32),
            jax.ShapeDtypeStruct((128, 1), jnp.int32),
        ),
        scratch_shapes=[pltpu.VMEM((_A, _E), jnp.float32)],
    )(x_TD, w_router_DE)

    sp = jnp.concatenate([be128[:_NB, 0][None, :], valid128[:_NB, 0][None, :]],
                         axis=0)
    posf = pos[:, 0]

    xs = _sc_dispatch(x_TD, toks[:, 0], posf)

    wg_bf = w_gate_EDF.astype(jnp.bfloat16)
    wu_bf = w_up_EDF.astype(jnp.bfloat16)
    wd_bf = w_down_EFD.astype(jnp.bfloat16)

    ys = pl.pallas_call(
        _expert_body,
        grid_spec=pltpu.PrefetchScalarGridSpec(
            num_scalar_prefetch=1,
            grid=(_NB,),
            in_specs=[
                pl.BlockSpec((_B, _D), lambda b, sp: (b, 0)),
                pl.BlockSpec((1, _D, _F), lambda b, sp: (sp[0, b], 0, 0)),
                pl.BlockSpec((1, _D, _F), lambda b, sp: (sp[0, b], 0, 0)),
                pl.BlockSpec((1, _F, _D), lambda b, sp: (sp[0, b], 0, 0)),
            ],
            out_specs=pl.BlockSpec((_B, _D), lambda b, sp: (b, 0)),
        ),
        out_shape=jax.ShapeDtypeStruct((_NPAD, _D), jnp.float32),
    )(sp, xs, wg_bf, wu_bf, wd_bf)

    out = _sc_combine(ys, posf, gates[:, 0])
    return out


# speedup vs baseline: 1.2412x; 1.0049x over previous
import functools

import jax
import jax.numpy as jnp
from jax import lax
from jax.experimental import pallas as pl
from jax.experimental.pallas import tpu as pltpu
from jax.experimental.pallas import tpu_sc as plsc

_T, _D, _F, _E, _K = 2048, 1024, 2048, 8, 2
_B = 512
_FC = 512
_A = _T * _K
_NB = _A // _B + _E - 1
_NPAD = _NB * _B

_NC = 2
_NS = 16
_NW = _NC * _NS
_APW = _A // _NW
_GCH = 64
_TPW = _T // _NW
_CCH = 32
_L = 16

_sc_mesh = plsc.VectorSubcoreMesh(core_axis_name="c", subcore_axis_name="s")


def _router_body(x_ref, wr_ref, pos_ref, gates_ref, toks_ref, be_ref, valid_ref,
                 oh_ref):
    x = x_ref[...]
    logits = jnp.dot(x, wr_ref[...], preferred_element_type=jnp.float32)
    iota = jax.lax.broadcasted_iota(jnp.int32, logits.shape, 1)
    m1 = jnp.max(logits, axis=-1, keepdims=True)
    i1 = jnp.argmax(logits, axis=-1)[:, None]
    masked = jnp.where(iota == i1, -jnp.inf, logits)
    m2 = jnp.max(masked, axis=-1, keepdims=True)
    i2 = jnp.argmax(masked, axis=-1)[:, None]
    z = jnp.exp(m2 - m1)
    g1 = 1.0 / (1.0 + z)
    g2 = z / (1.0 + z)
    oh1 = (iota == i1).astype(jnp.float32)
    oh2 = (iota == i2).astype(jnp.float32)
    oh_ref[0:_T, :] = oh1
    oh_ref[_T:_A, :] = oh2
    gates_ref[0:_T, :] = g1
    gates_ref[_T:_A, :] = g2
    toks_ref[...] = jax.lax.broadcasted_iota(jnp.int32, (_A, 1), 0) % _T

    n_e = jnp.sum(oh1, axis=0, keepdims=True) + jnp.sum(oh2, axis=0, keepdims=True)
    pc = jnp.ceil(n_e * (1.0 / _B)) * float(_B)
    eiota = jax.lax.broadcasted_iota(jnp.int32, (_E, _E), 0)
    ejota = jax.lax.broadcasted_iota(jnp.int32, (_E, _E), 1)
    strict_upper = (eiota < ejota).astype(jnp.float32)
    ps = jnp.dot(pc, strict_upper, preferred_element_type=jnp.float32)

    ri = jax.lax.broadcasted_iota(jnp.int32, (_B, _B), 0)
    ci = jax.lax.broadcasted_iota(jnp.int32, (_B, _B), 1)
    tri_s = (ci < ri).astype(jnp.float32)

    def body(c, run):
        ohc = oh_ref[pl.ds(c * _B, _B), :]
        local = jnp.dot(tri_s, ohc, preferred_element_type=jnp.float32)
        slot = jnp.sum((ps + run + local) * ohc, axis=1, keepdims=True)
        pos_ref[pl.ds(c * _B, _B), :] = slot.astype(jnp.int32)
        return run + jnp.sum(ohc, axis=0, keepdims=True)

    jax.lax.fori_loop(0, _A // _B, body, jnp.zeros((1, _E), jnp.float32))

    bs = jax.lax.broadcasted_iota(jnp.int32, (128, 1), 0).astype(jnp.float32) * float(_B)
    cnt = jnp.dot((bs >= ps).astype(jnp.float32), jnp.ones((_E, 1), jnp.float32),
                  preferred_element_type=jnp.float32)
    be_ref[...] = (cnt - 1.0).astype(jnp.int32)
    total_pad = jnp.sum(pc)
    valid_ref[...] = (bs < total_pad).astype(jnp.int32)


@functools.partial(
    pl.kernel,
    mesh=_sc_mesh,
    out_type=jax.ShapeDtypeStruct((_NPAD, _D), jnp.float32),
    scratch_types=[
        pltpu.VMEM((_GCH,), jnp.int32),
        pltpu.VMEM((_GCH,), jnp.int32),
        pltpu.VMEM((_GCH, _D), jnp.float32),
        pltpu.SemaphoreType.DMA,
    ],
)
def _sc_dispatch(x_hbm, toks_hbm, pos_hbm, xs_hbm, idx_v, dst_v, rows_v, sem):
    wid = lax.axis_index("s") * _NC + lax.axis_index("c")
    for c in range(_APW // _GCH):
        base = wid * _APW + c * _GCH
        pltpu.sync_copy(toks_hbm.at[pl.ds(base, _GCH)], idx_v)
        pltpu.sync_copy(pos_hbm.at[pl.ds(base, _GCH)], dst_v)
        pltpu.async_copy(x_hbm.at[idx_v], rows_v, sem).wait()
        pltpu.async_copy(rows_v, xs_hbm.at[dst_v], sem).wait()


def _expert_body(sp_ref, xs_ref, wg_ref, wu_ref, wd_ref, ys_ref):
    b = pl.program_id(0)

    @pl.when(sp_ref[1, b] == 1)
    def _compute():
        x = xs_ref[...].astype(jnp.bfloat16)
        acc = None
        for fc in range(_F // _FC):
            wg = wg_ref[0, :, fc * _FC:(fc + 1) * _FC]
            wu = wu_ref[0, :, fc * _FC:(fc + 1) * _FC]
            wd = wd_ref[0, fc * _FC:(fc + 1) * _FC, :]
            g = jnp.dot(x, wg, preferred_element_type=jnp.float32)
            u = jnp.dot(x, wu, preferred_element_type=jnp.float32)
            h = (g * jax.lax.logistic(g)) * u
            y = jnp.dot(h.astype(jnp.bfloat16), wd,
                        preferred_element_type=jnp.float32)
            acc = y if acc is None else acc + y
        ys_ref[...] = acc


@functools.partial(
    pl.kernel,
    mesh=_sc_mesh,
    out_type=jax.ShapeDtypeStruct((_T, _D), jnp.float32),
    scratch_types=[
        pltpu.VMEM((_CCH,), jnp.int32),
        pltpu.VMEM((_CCH,), jnp.int32),
        pltpu.VMEM((_CCH + _L,), jnp.float32),
        pltpu.VMEM((_CCH + _L,), jnp.float32),
        pltpu.VMEM((_CCH, _D), jnp.float32),
        pltpu.VMEM((_CCH, _D), jnp.float32),
        pltpu.SemaphoreType.DMA,
    ],
)
def _sc_combine(ys_hbm, pos_hbm, gates_hbm, out_hbm,
                p0_v, p1_v, g0_v, g1_v, buf0, buf1, sem):
    wid = lax.axis_index("s") * _NC + lax.axis_index("c")
    for c in range(_TPW // _CCH):
        tbase = wid * _TPW + c * _CCH
        pltpu.sync_copy(pos_hbm.at[pl.ds(tbase, _CCH)], p0_v)
        pltpu.sync_copy(pos_hbm.at[pl.ds(_T + tbase, _CCH)], p1_v)
        pltpu.sync_copy(gates_hbm.at[pl.ds(tbase, _CCH)], g0_v.at[pl.ds(0, _CCH)])
        pltpu.sync_copy(gates_hbm.at[pl.ds(_T + tbase, _CCH)], g1_v.at[pl.ds(0, _CCH)])
        pltpu.async_copy(ys_hbm.at[p0_v], buf0, sem).wait()
        pltpu.async_copy(ys_hbm.at[p1_v], buf1, sem).wait()

        def row_body(r, carry):
            ga = g0_v[pl.ds(r, _L)][0]
            gb = g1_v[pl.ds(r, _L)][0]

            def col_body(j, carry2):
                a = buf0[r, pl.ds(j * _L, _L)]
                b = buf1[r, pl.ds(j * _L, _L)]
                buf0[r, pl.ds(j * _L, _L)] = ga * a + gb * b
                return carry2

            return lax.fori_loop(0, _D // _L, col_body, carry)

        lax.fori_loop(0, _CCH, row_body, 0)
        pltpu.sync_copy(buf0, out_hbm.at[pl.ds(tbase, _CCH)])


def kernel(x_TD, w_router_DE, w_gate_EDF, w_up_EDF, w_down_EFD):
    pos, gates, toks, be128, valid128 = pl.pallas_call(
        _router_body,
        out_shape=(
            jax.ShapeDtypeStruct((_A, 1), jnp.int32),
            jax.ShapeDtypeStruct((_A, 1), jnp.float32),
            jax.ShapeDtypeStruct((_A, 1), jnp.int32),
            jax.ShapeDtypeStruct((128, 1), jnp.int32),
            jax.ShapeDtypeStruct((128, 1), jnp.int32),
        ),
        scratch_shapes=[pltpu.VMEM((_A, _E), jnp.float32)],
    )(x_TD, w_router_DE)

    sp = jnp.concatenate([be128[:_NB, 0][None, :], valid128[:_NB, 0][None, :]],
                         axis=0)
    posf = pos[:, 0]

    xs = _sc_dispatch(x_TD, toks[:, 0], posf)

    wg_bf = w_gate_EDF.astype(jnp.bfloat16)
    wu_bf = w_up_EDF.astype(jnp.bfloat16)
    wd_bf = w_down_EFD.astype(jnp.bfloat16)

    ys = pl.pallas_call(
        _expert_body,
        grid_spec=pltpu.PrefetchScalarGridSpec(
            num_scalar_prefetch=1,
            grid=(_NB,),
            in_specs=[
                pl.BlockSpec((_B, _D), lambda b, sp: (b, 0)),
                pl.BlockSpec((1, _D, _F), lambda b, sp: (sp[0, b], 0, 0)),
                pl.BlockSpec((1, _D, _F), lambda b, sp: (sp[0, b], 0, 0)),
                pl.BlockSpec((1, _F, _D), lambda b, sp: (sp[0, b], 0, 0)),
            ],
            out_specs=pl.BlockSpec((_B, _D), lambda b, sp: (b, 0)),
        ),
        out_shape=jax.ShapeDtypeStruct((_NPAD, _D), jnp.float32),
    )(sp, xs, wg_bf, wu_bf, wd_bf)

    out = _sc_combine(ys, posf, gates[:, 0])
    return out
